# baseline (speedup 1.0000x reference)
import functools

import jax
import jax.numpy as jnp
from jax import lax
from jax.experimental import pallas as pl
from jax.experimental.pallas import tpu as pltpu
from jax.experimental.pallas import tpu_sc as plsc

B, T, D = 4096, 200, 64
V = 1_000_000
ROWS = B * T
NC, NS, L = 2, 16, 16
NW = NC * NS
PAIRS = V // 2
ROWS_PER_W = ROWS // NW
CHUNK = 128
NCH = ROWS_PER_W // CHUNK
IDX_ROWS = ROWS // CHUNK
NBUF = 4

_mesh = plsc.VectorSubcoreMesh(core_axis_name="c", subcore_axis_name="s")


@functools.partial(
    pl.kernel,
    out_type=jax.ShapeDtypeStruct((ROWS // 2, 2 * D), jnp.float32),
    mesh=_mesh,
    scratch_types=[
        pltpu.VMEM((NBUF, CHUNK), jnp.int32),
        pltpu.VMEM((NBUF, CHUNK), jnp.int32),
        pltpu.VMEM((NBUF, CHUNK, 2 * D), jnp.float32),
        pltpu.VMEM((NBUF, CHUNK // 2, 2 * D), jnp.float32),
        pltpu.VMEM((T, D), jnp.float32),
        pltpu.SemaphoreType.DMA((NBUF,)),
        pltpu.SemaphoreType.DMA((NBUF,)),
    ],
    compiler_params=pltpu.CompilerParams(use_tc_tiling_on_sc=True),
)
def _sc_embed(idx_hbm, p_hbm, pos_hbm, out_hbm,
              idx_v, pid_v, rows_v, out_v, pos_v, g_sem, w_sem):
    wid = lax.axis_index("s") * NC + lax.axis_index("c")
    pltpu.sync_copy(pos_hbm, pos_v)
    out_base = wid * ROWS_PER_W
    idx_base = wid * NCH

    def issue_gather(c, b):
        pltpu.sync_copy(idx_hbm.at[idx_base + c, 0], idx_v.at[b])
        for g in range(CHUNK // L):
            sl = pl.ds(g * L, L)
            pid_v[b, sl] = lax.shift_right_logical(idx_v[b, sl], 1)
        pltpu.async_copy(p_hbm.at[pid_v.at[b]], rows_v.at[b], g_sem.at[b])

    def wait_gather(b):
        pltpu.make_async_copy(
            p_hbm.at[pl.ds(0, CHUNK)], rows_v.at[b], g_sem.at[b]
        ).wait()

    def compact_add(c, b):
        t0 = c * CHUNK

        @plsc.parallel_loop(0, CHUNK // L, 1)
        def _(k):
            r0 = k * L
            cbs = lax.shift_left(jnp.bitwise_and(idx_v[b, pl.ds(r0, L)], 1), 6)
            for l in range(L):
                r = r0 + l
                cb = cbs[l]
                q = r // 2
                ob = (l % 2) * D
                t = lax.rem(t0 + r, T)
                for g in range(D // L):
                    out_v[b, q, pl.ds(ob + g * L, L)] = (
                        rows_v[b, r, pl.ds(cb + g * L, L)]
                        + pos_v[t, pl.ds(g * L, L)]
                    )

    def issue_wb(c, b):
        off = pl.multiple_of((out_base + c * CHUNK) // 2, CHUNK // 2)
        pltpu.async_copy(
            out_v.at[b], out_hbm.at[pl.ds(off, CHUNK // 2)], w_sem.at[b]
        )

    def wait_wb(b):
        pltpu.make_async_copy(
            out_v.at[b], out_hbm.at[pl.ds(0, CHUNK // 2)], w_sem.at[b]
        ).wait()

    def finish(c, b):
        wait_gather(b)
        compact_add(c, b)
        issue_wb(c, b)

    for c in range(NBUF - 1):
        issue_gather(c, c)

    def outer(i, carry):
        for b2 in range(NBUF):
            c = i * NBUF + b2
            nb = (b2 + NBUF - 1) % NBUF

            @pl.when(c + NBUF - 1 < NCH)
            def _():
                @pl.when(c >= 1)
                def _():
                    wait_wb(nb)

                issue_gather(c + NBUF - 1, nb)

            finish(c, b2)
        return carry

    lax.fori_loop(0, NCH // NBUF, outer, 0)
    for b in range(NBUF):
        wait_wb(b)


def kernel(x, token_table, pos_table):
    idx = x.astype(jnp.int32).reshape(IDX_ROWS, 1, CHUNK)
    pairs = token_table.reshape(PAIRS, 2 * D)
    out = _sc_embed(idx, pairs, pos_table)
    return out.reshape(B, T, D)


# speedup vs baseline: 1.1088x; 1.1088x over previous
import functools

import jax
import jax.numpy as jnp
from jax import lax
from jax.experimental import pallas as pl
from jax.experimental.pallas import tpu as pltpu
from jax.experimental.pallas import tpu_sc as plsc

B, T, D = 4096, 200, 64
V = 1_000_000
ROWS = B * T
NC, NS, L = 2, 16, 16
NW = NC * NS
PAIRS = V // 2
ROWS_PER_W = ROWS // NW
CHUNK = 128
NCH = ROWS_PER_W // CHUNK
IDX_ROWS = ROWS // CHUNK
NBUF = 3

_mesh = plsc.VectorSubcoreMesh(core_axis_name="c", subcore_axis_name="s")


@functools.partial(
    pl.kernel,
    out_type=jax.ShapeDtypeStruct((ROWS // 2, 2 * D), jnp.float32),
    mesh=_mesh,
    scratch_types=[
        pltpu.VMEM((NCH, CHUNK), jnp.int32),
        pltpu.VMEM((NBUF, CHUNK), jnp.int32),
        pltpu.VMEM((NBUF, CHUNK, 2 * D), jnp.float32),
        pltpu.VMEM((NBUF, CHUNK // 2, 2 * D), jnp.float32),
        pltpu.VMEM((T, D), jnp.float32),
        pltpu.SemaphoreType.DMA((NBUF,)),
        pltpu.SemaphoreType.DMA((NBUF,)),
    ],
    compiler_params=pltpu.CompilerParams(use_tc_tiling_on_sc=True),
)
def _sc_embed(idx_hbm, p_hbm, pos_hbm, out_hbm,
              idx_v, pid_v, rows_v, out_v, pos_v, g_sem, w_sem):
    wid = lax.axis_index("s") * NC + lax.axis_index("c")
    pltpu.sync_copy(pos_hbm, pos_v)
    ib = pl.multiple_of(wid * NCH, 8)
    pltpu.sync_copy(idx_hbm.at[pl.ds(ib, NCH)], idx_v)
    out_base = wid * ROWS_PER_W

    def issue_gather(c, b):
        for g in range(CHUNK // L):
            sl = pl.ds(g * L, L)
            pid_v[b, sl] = lax.shift_right_logical(idx_v[c, sl], 1)
        pltpu.async_copy(p_hbm.at[pid_v.at[b]], rows_v.at[b], g_sem.at[b])

    def wait_gather(b):
        pltpu.make_async_copy(
            p_hbm.at[pl.ds(0, CHUNK)], rows_v.at[b], g_sem.at[b]
        ).wait()

    def compact_add(c, b):
        t0 = c * CHUNK

        @plsc.parallel_loop(0, CHUNK // L, 1)
        def _(k):
            r0 = k * L
            cbs = lax.shift_left(jnp.bitwise_and(idx_v[c, pl.ds(r0, L)], 1), 6)
            for l in range(L):
                r = r0 + l
                cb = cbs[l]
                q = r // 2
                ob = (l % 2) * D
                t = lax.rem(t0 + r, T)
                for g in range(D // L):
                    out_v[b, q, pl.ds(ob + g * L, L)] = (
                        rows_v[b, r, pl.ds(cb + g * L, L)]
                        + pos_v[t, pl.ds(g * L, L)]
                    )

    def issue_wb(c, b):
        off = pl.multiple_of((out_base + c * CHUNK) // 2, CHUNK // 2)
        pltpu.async_copy(
            out_v.at[b], out_hbm.at[pl.ds(off, CHUNK // 2)], w_sem.at[b]
        )

    def wait_wb(b):
        pltpu.make_async_copy(
            out_v.at[b], out_hbm.at[pl.ds(0, CHUNK // 2)], w_sem.at[b]
        ).wait()

    def finish(c, b):
        wait_gather(b)
        compact_add(c, b)
        issue_wb(c, b)

    for c in range(NBUF - 1):
        issue_gather(c, c)

    def outer(i, carry):
        for b2 in range(NBUF):
            c = i * NBUF + b2
            nb = (b2 + NBUF - 1) % NBUF

            @pl.when(c + NBUF - 1 < NCH)
            def _():
                @pl.when(c >= 1)
                def _():
                    wait_wb(nb)

                issue_gather(c + NBUF - 1, nb)

            @pl.when(c < NCH)
            def _():
                finish(c, b2)
        return carry

    lax.fori_loop(0, (NCH + NBUF - 1) // NBUF, outer, 0)
    for b in range(NBUF):
        wait_wb(b)


def kernel(x, token_table, pos_table):
    idx = x.astype(jnp.int32).reshape(IDX_ROWS, CHUNK)
    pairs = token_table.reshape(PAIRS, 2 * D)
    out = _sc_embed(idx, pairs, pos_table)
    return out.reshape(B, T, D)
